# Initial kernel scaffold; baseline (speedup 1.0000x reference)
import jax
import jax.numpy as jnp
from jax.experimental import pallas as pl


def kernel(hidden, edge_features, edge_sources, edge_targets, W_e, b):
    raise NotImplementedError("write your pallas kernel here")


# speedup vs baseline: 11.2196x; 11.2196x over previous
import functools

import jax
import jax.numpy as jnp
from jax import lax
from jax.experimental import pallas as pl
from jax.experimental.pallas import tpu as pltpu
from jax.experimental.pallas import tpu_sc as plsc

_LANES = 16
_N_SUBCORES = 16
_CHUNK = 80


def _edge_proj_kernel(ef_ref, w_ref, b_ref, o_ref):
    o_ref[...] = (
        jnp.dot(ef_ref[...], w_ref[...], preferred_element_type=jnp.float32)
        + b_ref[...]
    )


def _edge_proj(ef, w, b2d, blk):
    be, d = ef.shape
    m = w.shape[1]
    return pl.pallas_call(
        _edge_proj_kernel,
        grid=(be // blk,),
        in_specs=[
            pl.BlockSpec((blk, d), lambda i: (i, 0)),
            pl.BlockSpec((d, m), lambda i: (0, 0)),
            pl.BlockSpec((1, m), lambda i: (0, 0)),
        ],
        out_specs=pl.BlockSpec((blk, m), lambda i: (i, 0)),
        out_shape=jax.ShapeDtypeStruct((be, m), jnp.float32),
    )(ef, w, b2d)


def _make_sc_mp(B, N_pad, E, M):
    edges_per_tile = E // _N_SUBCORES
    n_chunks = edges_per_tile // _CHUNK
    assert edges_per_tile % _CHUNK == 0
    rows_per_tile = N_pad // _N_SUBCORES
    assert N_pad % (_N_SUBCORES * 8) == 0
    mesh = plsc.VectorSubcoreMesh(core_axis_name="c", subcore_axis_name="s")

    @functools.partial(
        pl.kernel,
        out_type=jax.ShapeDtypeStruct((B, N_pad, M), jnp.float32),
        mesh=mesh,
        scratch_types=[
            pltpu.VMEM_SHARED((N_pad, M), jnp.float32),
            pltpu.VMEM((_CHUNK,), jnp.int32),
            pltpu.VMEM((_CHUNK,), jnp.int32),
            pltpu.VMEM((_CHUNK, M), jnp.float32),
            pltpu.VMEM((_CHUNK, M), jnp.float32),
            pltpu.SemaphoreType.DMA,
        ],
    )
    def k(hidden_hbm, ep_hbm, src_hbm, tgt_hbm, zeros_hbm, out_hbm,
          acc, src_v, tgt_v, neigh_v, msg_v, sem):
        c = lax.axis_index("c")
        s = lax.axis_index("s")
        b = c
        r0 = s * rows_per_tile
        pltpu.sync_copy(zeros_hbm.at[pl.ds(r0, rows_per_tile)],
                        acc.at[pl.ds(r0, rows_per_tile)])
        plsc.subcore_barrier()

        tile_base = b * E + s * edges_per_tile

        def chunk_body(ci, carry):
            q = tile_base + ci * _CHUNK
            pltpu.sync_copy(src_hbm.at[pl.ds(q, _CHUNK)], src_v)
            pltpu.sync_copy(tgt_hbm.at[pl.ds(q, _CHUNK)], tgt_v)
            gath = pltpu.async_copy(hidden_hbm.at[src_v], neigh_v, sem)
            pltpu.sync_copy(ep_hbm.at[pl.ds(q, _CHUNK)], msg_v)
            gath.wait()

            def row_body(r, rc):
                for j in range(M // _LANES):
                    sl = pl.ds(j * _LANES, _LANES)
                    msg_v[r, sl] = jnp.maximum(neigh_v[r, sl] + msg_v[r, sl], 0.0)
                return rc

            lax.fori_loop(0, _CHUNK, row_body, 0)
            pltpu.sync_copy(msg_v, acc.at[tgt_v], add=True)
            return carry

        lax.fori_loop(0, n_chunks, chunk_body, 0)
        plsc.subcore_barrier()
        pltpu.sync_copy(acc.at[pl.ds(r0, rows_per_tile)],
                        out_hbm.at[b, pl.ds(r0, rows_per_tile)])

    return k


def kernel(hidden, edge_features, edge_sources, edge_targets, W_e, b):
    B, N, H = hidden.shape
    _, E, D_E = edge_features.shape
    M = W_e.shape[1]
    src = edge_sources.astype(jnp.int32) + jnp.arange(B, dtype=jnp.int32)[:, None] * N
    tgt = edge_targets.astype(jnp.int32)
    ep = _edge_proj(edge_features.reshape(B * E, D_E), W_e,
                    b.reshape(1, M).astype(jnp.float32), blk=3200)
    n_pad = ((N + _N_SUBCORES * 8 - 1) // (_N_SUBCORES * 8)) * (_N_SUBCORES * 8)
    zeros = jnp.zeros((n_pad, M), jnp.float32)
    k = _make_sc_mp(B, n_pad, E, M)
    out = k(hidden.reshape(B * N, H), ep, src.reshape(-1), tgt.reshape(-1), zeros)
    return out[:, :N, :]


# speedup vs baseline: 14.3732x; 1.2811x over previous
import functools

import jax
import jax.numpy as jnp
from jax import lax
from jax.experimental import pallas as pl
from jax.experimental.pallas import tpu as pltpu
from jax.experimental.pallas import tpu_sc as plsc

_LANES = 16
_N_SUBCORES = 16
_CHUNK = 56
_GROUP = 4


def _edge_proj_kernel(ef_ref, w_ref, b_ref, o_ref):
    o_ref[...] = (
        jnp.dot(ef_ref[...], w_ref[...], preferred_element_type=jnp.float32)
        + b_ref[...]
    )


def _edge_proj(ef, w, b2d, blk):
    be, d = ef.shape
    m = w.shape[1]
    return pl.pallas_call(
        _edge_proj_kernel,
        grid=(be // blk,),
        in_specs=[
            pl.BlockSpec((blk, d), lambda i: (i, 0)),
            pl.BlockSpec((d, m), lambda i: (0, 0)),
            pl.BlockSpec((1, m), lambda i: (0, 0)),
        ],
        out_specs=pl.BlockSpec((blk, m), lambda i: (i, 0)),
        out_shape=jax.ShapeDtypeStruct((be, m), jnp.float32),
    )(ef, w, b2d)


def _make_sc_mp(B, N, N_pad, E_pad, M):
    edges_per_tile = E_pad // _N_SUBCORES
    n_chunks = edges_per_tile // _CHUNK
    assert edges_per_tile % (_CHUNK * _GROUP) == 0
    n_groups = n_chunks // _GROUP
    rows_per_tile = N_pad // _N_SUBCORES
    assert N_pad % (_N_SUBCORES * 8) == 0
    rows_last = N - rows_per_tile * (_N_SUBCORES - 1)
    assert rows_last % 8 == 0 and rows_last > 0
    mesh = plsc.VectorSubcoreMesh(core_axis_name="c", subcore_axis_name="s")

    @functools.partial(
        pl.kernel,
        out_type=jax.ShapeDtypeStruct((B, N, M), jnp.float32),
        mesh=mesh,
        scratch_types=[
            pltpu.VMEM_SHARED((N_pad, M), jnp.float32),
            [pltpu.VMEM((_CHUNK,), jnp.int32)] * 4,
            [pltpu.VMEM((_CHUNK,), jnp.int32)] * 4,
            [pltpu.VMEM((_CHUNK, M), jnp.float32)] * 2,
            [pltpu.VMEM((_CHUNK, M), jnp.float32)] * 2,
            [pltpu.VMEM((_CHUNK, M), jnp.float32)] * 2,
            [pltpu.SemaphoreType.DMA] * 4,
            [pltpu.SemaphoreType.DMA] * 4,
            [pltpu.SemaphoreType.DMA] * 2,
            [pltpu.SemaphoreType.DMA] * 2,
            [pltpu.SemaphoreType.DMA] * 2,
        ],
    )
    def k(hidden_hbm, ep_hbm, src_hbm, tgt_hbm, zeros_hbm, out_hbm,
          acc, src_v, tgt_v, neigh_v, epv, msg_v, ssem, tsem, gsem, esem, wsem):
        c = lax.axis_index("c")
        s = lax.axis_index("s")
        b = c
        r0 = s * rows_per_tile
        pltpu.sync_copy(zeros_hbm.at[pl.ds(r0, rows_per_tile)],
                        acc.at[pl.ds(r0, rows_per_tile)])
        plsc.subcore_barrier()

        tile_base = b * E_pad + s * edges_per_tile

        def issue_idx(ci, s4):
            q = tile_base + ci * _CHUNK
            pltpu.async_copy(src_hbm.at[pl.ds(q, _CHUNK)], src_v[s4], ssem[s4])
            pltpu.async_copy(tgt_hbm.at[pl.ds(q, _CHUNK)], tgt_v[s4], tsem[s4])

        def wait_idx(s4):
            pltpu.make_async_copy(src_hbm.at[pl.ds(0, _CHUNK)], src_v[s4],
                                  ssem[s4]).wait()
            pltpu.make_async_copy(tgt_hbm.at[pl.ds(0, _CHUNK)], tgt_v[s4],
                                  tsem[s4]).wait()

        def issue_data(ci, s4, p):
            q = tile_base + ci * _CHUNK
            pltpu.async_copy(hidden_hbm.at[src_v[s4]], neigh_v[p], gsem[p])
            pltpu.async_copy(ep_hbm.at[pl.ds(q, _CHUNK)], epv[p], esem[p])

        def wait_data(s4, p):
            pltpu.make_async_copy(hidden_hbm.at[src_v[s4]], neigh_v[p],
                                  gsem[p]).wait()
            pltpu.make_async_copy(ep_hbm.at[pl.ds(0, _CHUNK)], epv[p],
                                  esem[p]).wait()

        def wait_scatter(s4, p):
            pltpu.make_async_copy(msg_v[p], acc.at[tgt_v[s4]], wsem[p]).wait()

        def compute(p):
            nb, eb, mb = neigh_v[p], epv[p], msg_v[p]

            def row_body(r, rc):
                for j in range(M // _LANES):
                    sl = pl.ds(j * _LANES, _LANES)
                    mb[r, sl] = jnp.maximum(nb[r, sl] + eb[r, sl], 0.0)
                return rc

            lax.fori_loop(0, _CHUNK, row_body, 0)

        issue_idx(0, 0)
        issue_idx(1, 1)
        wait_idx(0)
        issue_data(0, 0, 0)

        def group_body(g, carry):
            for j in range(_GROUP):
                ci = g * _GROUP + j
                p = j % 2

                @pl.when(ci + 1 < n_chunks)
                def _():
                    wait_idx((j + 1) % 4)
                    issue_data(ci + 1, (j + 1) % 4, (j + 1) % 2)

                wait_data(j, p)

                @pl.when(ci >= 2)
                def _():
                    wait_scatter((j + 2) % 4, p)

                compute(p)
                pltpu.async_copy(msg_v[p], acc.at[tgt_v[j]], wsem[p], add=True)

                @pl.when(ci + 2 < n_chunks)
                def _():
                    issue_idx(ci + 2, (j + 2) % 4)
            return carry

        lax.fori_loop(0, n_groups, group_body, 0)
        wait_scatter((n_chunks - 2) % 4, (n_chunks - 2) % 2)
        wait_scatter((n_chunks - 1) % 4, (n_chunks - 1) % 2)
        plsc.subcore_barrier()

        @pl.when(s < _N_SUBCORES - 1)
        def _():
            pltpu.sync_copy(acc.at[pl.ds(r0, rows_per_tile)],
                            out_hbm.at[b, pl.ds(r0, rows_per_tile)])

        @pl.when(s == _N_SUBCORES - 1)
        def _():
            q = (_N_SUBCORES - 1) * rows_per_tile
            pltpu.sync_copy(acc.at[pl.ds(q, rows_last)],
                            out_hbm.at[b, pl.ds(q, rows_last)])

    return k


def kernel(hidden, edge_features, edge_sources, edge_targets, W_e, b):
    B, N, H = hidden.shape
    _, E, D_E = edge_features.shape
    M = W_e.shape[1]

    n_pad = ((N + _N_SUBCORES * 8 - 1) // (_N_SUBCORES * 8)) * (_N_SUBCORES * 8)
    step = _N_SUBCORES * _CHUNK * _GROUP
    e_pad = ((E + step - 1) // step) * step
    npad_e = e_pad - E

    src = edge_sources.astype(jnp.int32) + jnp.arange(B, dtype=jnp.int32)[:, None] * N
    tgt = edge_targets.astype(jnp.int32)
    pad_src = (jnp.arange(npad_e, dtype=jnp.int32) % N)[None, :] \
        + jnp.arange(B, dtype=jnp.int32)[:, None] * N
    pad_tgt = jnp.broadcast_to(
        N + jnp.arange(npad_e, dtype=jnp.int32) % (n_pad - N), (B, npad_e))
    src = jnp.concatenate([src, pad_src], axis=1)
    tgt = jnp.concatenate([tgt, pad_tgt], axis=1)
    ef = jnp.concatenate(
        [edge_features,
         jnp.zeros((B, npad_e, D_E), edge_features.dtype)], axis=1)

    blk = next(d for d in range(4096, 7, -8) if (B * e_pad) % d == 0)
    ep = _edge_proj(ef.reshape(B * e_pad, D_E), W_e,
                    b.reshape(1, M).astype(jnp.float32), blk=blk)
    zeros = jnp.zeros((n_pad, M), jnp.float32)
    k = _make_sc_mp(B, N, n_pad, e_pad, M)
    return k(hidden.reshape(B * N, H), ep, src.reshape(-1), tgt.reshape(-1), zeros)


# speedup vs baseline: 18.6717x; 1.2991x over previous
import functools

import jax
import jax.numpy as jnp
from jax import lax
from jax.experimental import pallas as pl
from jax.experimental.pallas import tpu as pltpu
from jax.experimental.pallas import tpu_sc as plsc

_LANES = 16
_N_SUBCORES = 16
_CHUNK = 56
_GROUP = 4


def _edge_proj_kernel(ef_ref, w_ref, b_ref, o_ref):
    o_ref[...] = (
        jnp.dot(ef_ref[...], w_ref[...], preferred_element_type=jnp.float32)
        + b_ref[...]
    )


def _edge_proj(ef, w, b2d, blk):
    be, d = ef.shape
    m = w.shape[1]
    return pl.pallas_call(
        _edge_proj_kernel,
        grid=(be // blk,),
        in_specs=[
            pl.BlockSpec((blk, d), lambda i: (i, 0)),
            pl.BlockSpec((d, m), lambda i: (0, 0)),
            pl.BlockSpec((1, m), lambda i: (0, 0)),
        ],
        out_specs=pl.BlockSpec((blk, m), lambda i: (i, 0)),
        out_shape=jax.ShapeDtypeStruct((be, m), jnp.float32),
    )(ef, w, b2d)


def _make_sc_mp(B, N, N_pad, E, M):
    edges_per_tile = E // _N_SUBCORES
    n_pipe = (edges_per_tile // _CHUNK // _GROUP) * _GROUP
    n_groups = n_pipe // _GROUP
    tail = edges_per_tile - n_pipe * _CHUNK
    n_tail_full = tail // _CHUNK
    rem = tail - n_tail_full * _CHUNK
    assert rem % 8 == 0
    rows_per_tile = N_pad // _N_SUBCORES
    assert N_pad % (_N_SUBCORES * 8) == 0
    rows_last = N - rows_per_tile * (_N_SUBCORES - 1)
    assert rows_last % 8 == 0 and rows_last > 0
    zrep = rows_per_tile // _CHUNK
    zrem = rows_per_tile - zrep * _CHUNK
    assert zrem % 8 == 0
    mesh = plsc.VectorSubcoreMesh(core_axis_name="c", subcore_axis_name="s")

    @functools.partial(
        pl.kernel,
        out_type=jax.ShapeDtypeStruct((B, N, M), jnp.float32),
        mesh=mesh,
        scratch_types=[
            pltpu.VMEM_SHARED((N_pad, M), jnp.float32),
            [pltpu.VMEM((_CHUNK,), jnp.int32)] * 4,
            [pltpu.VMEM((_CHUNK,), jnp.int32)] * 4,
            pltpu.VMEM((max(rem, 8),), jnp.int32),
            pltpu.VMEM((max(rem, 8),), jnp.int32),
            [pltpu.VMEM((_CHUNK, M), jnp.float32)] * 2,
            [pltpu.VMEM((_CHUNK, M), jnp.float32)] * 2,
            [pltpu.VMEM((_CHUNK, M), jnp.float32)] * 2,
            [pltpu.SemaphoreType.DMA] * 4,
            [pltpu.SemaphoreType.DMA] * 4,
            [pltpu.SemaphoreType.DMA] * 2,
            [pltpu.SemaphoreType.DMA] * 2,
            [pltpu.SemaphoreType.DMA] * 2,
        ],
    )
    def k(hidden_hbm, ep_hbm, src_hbm, tgt_hbm, out_hbm,
          acc, src_v, tgt_v, src_t, tgt_t, neigh_v, epv, msg_v,
          ssem, tsem, gsem, esem, wsem):
        c = lax.axis_index("c")
        s = lax.axis_index("s")
        b = c
        r0 = s * rows_per_tile

        def zrow(r, rc):
            zv = jnp.zeros((_LANES,), jnp.float32)
            for j in range(M // _LANES):
                msg_v[1][r, pl.ds(j * _LANES, _LANES)] = zv
            return rc

        lax.fori_loop(0, _CHUNK, zrow, 0)
        for t in range(zrep):
            pltpu.sync_copy(msg_v[1], acc.at[pl.ds(r0 + t * _CHUNK, _CHUNK)])
        if zrem:
            pltpu.sync_copy(msg_v[1].at[pl.ds(0, zrem)],
                            acc.at[pl.ds(r0 + zrep * _CHUNK, zrem)])
        plsc.subcore_barrier()

        tile_base = b * E + s * edges_per_tile

        def issue_idx(ci, s4):
            q = tile_base + ci * _CHUNK
            pltpu.async_copy(src_hbm.at[pl.ds(q, _CHUNK)], src_v[s4], ssem[s4])
            pltpu.async_copy(tgt_hbm.at[pl.ds(q, _CHUNK)], tgt_v[s4], tsem[s4])

        def wait_idx(s4):
            pltpu.make_async_copy(src_hbm.at[pl.ds(0, _CHUNK)], src_v[s4],
                                  ssem[s4]).wait()
            pltpu.make_async_copy(tgt_hbm.at[pl.ds(0, _CHUNK)], tgt_v[s4],
                                  tsem[s4]).wait()

        def issue_data(ci, s4, p):
            q = tile_base + ci * _CHUNK
            pltpu.async_copy(hidden_hbm.at[src_v[s4]], neigh_v[p], gsem[p])
            pltpu.async_copy(ep_hbm.at[pl.ds(q, _CHUNK)], epv[p], esem[p])

        def wait_data(s4, p):
            pltpu.make_async_copy(hidden_hbm.at[src_v[s4]], neigh_v[p],
                                  gsem[p]).wait()
            pltpu.make_async_copy(ep_hbm.at[pl.ds(0, _CHUNK)], epv[p],
                                  esem[p]).wait()

        def wait_scatter(s4, p):
            pltpu.make_async_copy(msg_v[p], acc.at[tgt_v[s4]], wsem[p]).wait()

        def compute(p, nrows):
            nb, eb, mb = neigh_v[p], epv[p], msg_v[p]

            def row_body(r, rc):
                for j in range(M // _LANES):
                    sl = pl.ds(j * _LANES, _LANES)
                    mb[r, sl] = jnp.maximum(nb[r, sl] + eb[r, sl], 0.0)
                return rc

            lax.fori_loop(0, nrows, row_body, 0)

        issue_idx(0, 0)
        issue_idx(1, 1)
        wait_idx(0)
        issue_data(0, 0, 0)

        def group_body(g, carry):
            for j in range(_GROUP):
                ci = g * _GROUP + j
                p = j % 2

                @pl.when(ci + 1 < n_pipe)
                def _():
                    wait_idx((j + 1) % 4)
                    issue_data(ci + 1, (j + 1) % 4, (j + 1) % 2)

                wait_data(j, p)

                @pl.when(ci >= 2)
                def _():
                    wait_scatter((j + 2) % 4, p)

                compute(p, _CHUNK)
                pltpu.async_copy(msg_v[p], acc.at[tgt_v[j]], wsem[p], add=True)

                @pl.when(ci + 2 < n_pipe)
                def _():
                    issue_idx(ci + 2, (j + 2) % 4)
            return carry

        lax.fori_loop(0, n_groups, group_body, 0)
        wait_scatter((n_pipe - 2) % 4, (n_pipe - 2) % 2)
        wait_scatter((n_pipe - 1) % 4, (n_pipe - 1) % 2)

        for t in range(n_tail_full):
            ci = n_pipe + t
            q = tile_base + ci * _CHUNK
            pltpu.sync_copy(src_hbm.at[pl.ds(q, _CHUNK)], src_v[0])
            pltpu.sync_copy(tgt_hbm.at[pl.ds(q, _CHUNK)], tgt_v[0])
            pltpu.async_copy(hidden_hbm.at[src_v[0]], neigh_v[0], gsem[0]).wait()
            pltpu.sync_copy(ep_hbm.at[pl.ds(q, _CHUNK)], epv[0])
            compute(0, _CHUNK)
            pltpu.sync_copy(msg_v[0], acc.at[tgt_v[0]], add=True)

        if rem:
            q = tile_base + (n_pipe + n_tail_full) * _CHUNK
            pltpu.sync_copy(src_hbm.at[pl.ds(q, rem)], src_t)
            pltpu.sync_copy(tgt_hbm.at[pl.ds(q, rem)], tgt_t)
            nv = neigh_v[1].at[pl.ds(0, rem)]
            ev = epv[1].at[pl.ds(0, rem)]
            pltpu.async_copy(hidden_hbm.at[src_t], nv, gsem[1]).wait()
            pltpu.sync_copy(ep_hbm.at[pl.ds(q, rem)], ev)
            compute(1, rem)
            pltpu.sync_copy(msg_v[1].at[pl.ds(0, rem)], acc.at[tgt_t], add=True)

        plsc.subcore_barrier()

        @pl.when(s < _N_SUBCORES - 1)
        def _():
            pltpu.sync_copy(acc.at[pl.ds(r0, rows_per_tile)],
                            out_hbm.at[b, pl.ds(r0, rows_per_tile)])

        @pl.when(s == _N_SUBCORES - 1)
        def _():
            q = (_N_SUBCORES - 1) * rows_per_tile
            pltpu.sync_copy(acc.at[pl.ds(q, rows_last)],
                            out_hbm.at[b, pl.ds(q, rows_last)])

    return k


def kernel(hidden, edge_features, edge_sources, edge_targets, W_e, b):
    B, N, H = hidden.shape
    _, E, D_E = edge_features.shape
    M = W_e.shape[1]

    n_pad = ((N + _N_SUBCORES * 8 - 1) // (_N_SUBCORES * 8)) * (_N_SUBCORES * 8)
    src = edge_sources.astype(jnp.int32) + jnp.arange(B, dtype=jnp.int32)[:, None] * N
    tgt = edge_targets.astype(jnp.int32)

    blk = next(d for d in range(4096, 7, -8) if (B * E) % d == 0)
    ep = _edge_proj(edge_features.reshape(B * E, D_E), W_e,
                    b.reshape(1, M).astype(jnp.float32), blk=blk)
    k = _make_sc_mp(B, N, n_pad, E, M)
    return k(hidden.reshape(B * N, H), ep, src.reshape(-1), tgt.reshape(-1))
